# baseline (speedup 1.0000x reference)
import functools

import jax
import jax.numpy as jnp
from jax import lax
from jax.experimental import pallas as pl
from jax.experimental.pallas import tpu as pltpu
from jax.experimental.pallas import tpu_sc as plsc

VOCAB = 100000
EMBED_DIM = 256
NUM_LABELS = 2
BATCH = 1024
SEQ = 200
DPAD = 16

VBLK = 12800
NBLK = -(-VOCAB // VBLK)
VPAD = NBLK * VBLK
GRP = VBLK // 8
GSHIFT = (GRP // 25).bit_length() - 1
assert GRP == 25 << GSHIFT and (VPAD - 1) >> GSHIFT <= 1599


def _proj_body(tab_ref, w_ref, out_ref):
    mm2 = lax.dot_general(tab_ref[...], w_ref[...],
                          (((1,), (1,)), ((), ())),
                          preferred_element_type=jnp.float32)
    mm = jnp.concatenate(
        [mm2, jnp.zeros((VBLK, DPAD - NUM_LABELS), jnp.float32)], axis=1)
    out_ref[...] = jnp.concatenate(
        [mm[k * GRP:(k + 1) * GRP, :] for k in range(8)], axis=1)


def _project_table(embed_table, w2):
    return pl.pallas_call(
        _proj_body,
        grid=(NBLK,),
        in_specs=[
            pl.BlockSpec((VBLK, EMBED_DIM), lambda i: (i, 0)),
            pl.BlockSpec((NUM_LABELS, EMBED_DIM), lambda i: (0, 0)),
        ],
        out_specs=pl.BlockSpec((GRP, 8 * DPAD), lambda i: (i, 0)),
        out_shape=jax.ShapeDtypeStruct((NBLK * GRP, 8 * DPAD), jnp.float32),
    )(embed_table, w2)


def _make_pool_kernel():
    info = plsc.get_sparse_core_info()
    nw = info.num_cores * info.num_subcores
    b_per_w = BATCH // nw
    n_idx = b_per_w * SEQ
    halves = (104, 96)
    chunk = 128
    n_chunks = n_idx // chunk
    nsems = 10
    ring_rows = 3200
    ring_b = ring_rows // SEQ

    mesh = plsc.VectorSubcoreMesh(core_axis_name="c", subcore_axis_name="s")

    @functools.partial(
        pl.kernel,
        mesh=mesh,
        compiler_params=pltpu.CompilerParams(use_tc_tiling_on_sc=False),
        out_type=jax.ShapeDtypeStruct((BATCH, DPAD), jnp.float32),
        scratch_types=[
            pltpu.VMEM((n_chunks, chunk), jnp.int32),
            pltpu.VMEM((ring_rows, DPAD), jnp.float32),
            pltpu.VMEM((b_per_w, DPAD), jnp.float32),
        ] + [pltpu.SemaphoreType.DMA] * nsems,
    )
    def pool(idx_hbm, p_hbm, out_hbm, idx_v, ring_v, out_v, *sems):
        wid = lax.axis_index("s") * info.num_cores + lax.axis_index("c")
        pltpu.sync_copy(
            idx_hbm.at[pl.ds(pl.multiple_of(wid * n_chunks, n_chunks),
                             n_chunks), :], idx_v)

        def remap(c):
            for k in range(chunk // 16):
                v = idx_v[c, pl.ds(k * 16, 16)]
                q = ((v >> GSHIFT) * 1311) >> 15
                idx_v[c, pl.ds(k * 16, 16)] = (
                    ((q >> 3) * VBLK) + 8 * (v - q * GRP) + (q & 7))

        def ring_at(c):
            return pl.multiple_of((c - 25 * ((c * 1311) >> 15)) * chunk, chunk)

        def fire(c, slot):
            pltpu.async_copy(p_hbm.at[idx_v.at[c]],
                             ring_v.at[pl.ds(ring_at(c), chunk), :],
                             sems[slot])

        def drain(c, slot):
            pltpu.make_async_copy(p_hbm.at[pl.ds(0, chunk)],
                                  ring_v.at[pl.ds(ring_at(c), chunk), :],
                                  sems[slot]).wait()

        for slot in range(nsems):
            remap(slot)
            fire(slot, slot)

        inv = jnp.float32(1.0 / SEQ)
        zero = jnp.zeros((DPAD,), jnp.float32)

        def reduce_row(r, carry):
            base = 200 * (r & (ring_b - 1))

            def accum(t, accs):
                a0, a1, a2, a3 = accs
                o = base + t * 8
                a0 = a0 + ring_v[o + 0, :]
                a1 = a1 + ring_v[o + 1, :]
                a2 = a2 + ring_v[o + 2, :]
                a3 = a3 + ring_v[o + 3, :]
                a0 = a0 + ring_v[o + 4, :]
                a1 = a1 + ring_v[o + 5, :]
                a2 = a2 + ring_v[o + 6, :]
                a3 = a3 + ring_v[o + 7, :]
                return a0, a1, a2, a3

            a0, a1, a2, a3 = lax.fori_loop(0, SEQ // 8, accum,
                                           (zero, zero, zero, zero))
            out_v[r, :] = ((a0 + a1) + (a2 + a3)) * inv
            return carry

        def step(c, slot, done):
            drain(c, slot)

            @pl.when(c + nsems < n_chunks)
            def _():
                remap(c + nsems)
                fire(c + nsems, slot)

            new_done = ((16 * (c + 1)) * 1311) >> 15
            lax.fori_loop(done, new_done, reduce_row, 0)
            return new_done

        def rounds(u, done):
            for slot in range(nsems):
                done = step(u * nsems + slot, slot, done)
            return done

        done = lax.fori_loop(0, n_chunks // nsems, rounds, 0)
        for slot in range(n_chunks % nsems):
            done = step((n_chunks // nsems) * nsems + slot, slot, done)

        pltpu.sync_copy(
            out_v,
            out_hbm.at[pl.ds(pl.multiple_of(wid * b_per_w, b_per_w), b_per_w), :])

    return pool, nw


def kernel(indices, embed_table, W, b):
    p_packed = _project_table(embed_table, W)
    p = p_packed.reshape(VPAD, DPAD)

    pool, nw = _make_pool_kernel()
    out16 = pool(indices.reshape(-1, 128), p)
    return out16[:, :NUM_LABELS] + b


# speedup vs baseline: 1.0086x; 1.0086x over previous
import functools

import jax
import jax.numpy as jnp
from jax import lax
from jax.experimental import pallas as pl
from jax.experimental.pallas import tpu as pltpu
from jax.experimental.pallas import tpu_sc as plsc

VOCAB = 100000
EMBED_DIM = 256
NUM_LABELS = 2
BATCH = 1024
SEQ = 200
DPAD = 16

VBLK = 12800
NBLK = -(-VOCAB // VBLK)
VPAD = NBLK * VBLK
GRP = VBLK // 8
GSHIFT = (GRP // 25).bit_length() - 1
assert GRP == 25 << GSHIFT and (VPAD - 1) >> GSHIFT <= 1599


def _proj_body(tab_ref, w_ref, out_ref):
    mm2 = lax.dot_general(tab_ref[...], w_ref[...],
                          (((1,), (1,)), ((), ())),
                          preferred_element_type=jnp.float32)
    mm = jnp.concatenate(
        [mm2, jnp.zeros((VBLK, DPAD - NUM_LABELS), jnp.float32)], axis=1)
    out_ref[...] = jnp.concatenate(
        [mm[k * GRP:(k + 1) * GRP, :] for k in range(8)], axis=1)


def _project_table(embed_table, w2):
    return pl.pallas_call(
        _proj_body,
        grid=(NBLK,),
        in_specs=[
            pl.BlockSpec((VBLK, EMBED_DIM), lambda i: (i, 0)),
            pl.BlockSpec((NUM_LABELS, EMBED_DIM), lambda i: (0, 0)),
        ],
        out_specs=pl.BlockSpec((GRP, 8 * DPAD), lambda i: (i, 0)),
        out_shape=jax.ShapeDtypeStruct((NBLK * GRP, 8 * DPAD), jnp.float32),
    )(embed_table, w2)


def _make_pool_kernel():
    info = plsc.get_sparse_core_info()
    nw = info.num_cores * info.num_subcores
    b_per_w = BATCH // nw
    n_idx = b_per_w * SEQ
    halves = (104, 96)
    chunk = 128
    n_chunks = n_idx // chunk
    nsems = 8
    ring_rows = 3200
    ring_b = ring_rows // SEQ

    mesh = plsc.VectorSubcoreMesh(core_axis_name="c", subcore_axis_name="s")

    @functools.partial(
        pl.kernel,
        mesh=mesh,
        compiler_params=pltpu.CompilerParams(use_tc_tiling_on_sc=False),
        out_type=jax.ShapeDtypeStruct((BATCH, DPAD), jnp.float32),
        scratch_types=[
            pltpu.VMEM((n_chunks, chunk), jnp.int32),
            pltpu.VMEM((ring_rows, DPAD), jnp.float32),
            pltpu.VMEM((b_per_w, DPAD), jnp.float32),
        ] + [pltpu.SemaphoreType.DMA] * nsems,
    )
    def pool(idx_hbm, p_hbm, out_hbm, idx_v, ring_v, out_v, *sems):
        wid = lax.axis_index("s") * info.num_cores + lax.axis_index("c")
        pltpu.sync_copy(
            idx_hbm.at[pl.ds(pl.multiple_of(wid * n_chunks, n_chunks),
                             n_chunks), :], idx_v)

        def remap(c):
            for k in range(chunk // 16):
                v = idx_v[c, pl.ds(k * 16, 16)]
                q = ((v >> GSHIFT) * 1311) >> 15
                idx_v[c, pl.ds(k * 16, 16)] = (
                    ((q >> 3) * VBLK) + 8 * (v - q * GRP) + (q & 7))

        def ring_at(c):
            return pl.multiple_of((c - 25 * ((c * 1311) >> 15)) * chunk, chunk)

        def fire(c, slot):
            pltpu.async_copy(p_hbm.at[idx_v.at[c]],
                             ring_v.at[pl.ds(ring_at(c), chunk), :],
                             sems[slot])

        def drain(c, slot):
            pltpu.make_async_copy(p_hbm.at[pl.ds(0, chunk)],
                                  ring_v.at[pl.ds(ring_at(c), chunk), :],
                                  sems[slot]).wait()

        for slot in range(nsems):
            remap(slot)
            fire(slot, slot)

        inv = jnp.float32(1.0 / SEQ)
        zero = jnp.zeros((DPAD,), jnp.float32)

        def reduce_row(r, carry):
            base = 200 * (r & (ring_b - 1))

            def accum(t, accs):
                a0, a1, a2, a3 = accs
                o = base + t * 8
                a0 = a0 + ring_v[o + 0, :]
                a1 = a1 + ring_v[o + 1, :]
                a2 = a2 + ring_v[o + 2, :]
                a3 = a3 + ring_v[o + 3, :]
                a0 = a0 + ring_v[o + 4, :]
                a1 = a1 + ring_v[o + 5, :]
                a2 = a2 + ring_v[o + 6, :]
                a3 = a3 + ring_v[o + 7, :]
                return a0, a1, a2, a3

            a0, a1, a2, a3 = lax.fori_loop(0, SEQ // 8, accum,
                                           (zero, zero, zero, zero))
            out_v[r, :] = ((a0 + a1) + (a2 + a3)) * inv
            return carry

        def step(c, slot, done):
            drain(c, slot)

            @pl.when(c + nsems < n_chunks)
            def _():
                remap(c + nsems)
                fire(c + nsems, slot)

            new_done = ((16 * (c + 1)) * 1311) >> 15
            lax.fori_loop(done, new_done, reduce_row, 0)
            return new_done

        def rounds(u, done):
            for slot in range(nsems):
                done = step(u * nsems + slot, slot, done)
            return done

        done = lax.fori_loop(0, n_chunks // nsems, rounds, 0)
        for slot in range(n_chunks % nsems):
            done = step((n_chunks // nsems) * nsems + slot, slot, done)

        pltpu.sync_copy(
            out_v,
            out_hbm.at[pl.ds(pl.multiple_of(wid * b_per_w, b_per_w), b_per_w), :])

    return pool, nw


def kernel(indices, embed_table, W, b):
    p_packed = _project_table(embed_table, W)
    p = p_packed.reshape(VPAD, DPAD)

    pool, nw = _make_pool_kernel()
    out16 = pool(indices.reshape(-1, 128), p)
    return out16[:, :NUM_LABELS] + b
